# Initial kernel scaffold; baseline (speedup 1.0000x reference)
import jax
import jax.numpy as jnp
from jax.experimental import pallas as pl


def kernel(x, edge_index, W, b):
    raise NotImplementedError("write your pallas kernel here")


# speedup vs baseline: 12.6348x; 12.6348x over previous
import functools

import jax
import jax.numpy as jnp
from jax import lax
from jax.experimental import pallas as pl
from jax.experimental.pallas import tpu as pltpu
from jax.experimental.pallas import tpu_sc as plsc

N = 10000
C = 128
E = 320000

NC = 2
NS = 16
NW = NC * NS

PAD_N = 10112
E_PAD = 327680
CHUNK = 128
EPT = E_PAD // NW
NCHUNK = EPT // CHUNK
RPT = PAD_N // NS
DEG_PAD = 10240
DPT = DEG_PAD // NS

_MESH = plsc.VectorSubcoreMesh(core_axis_name="c", subcore_axis_name="s")


@functools.partial(
    pl.kernel,
    mesh=_MESH,
    out_type=jax.ShapeDtypeStruct((NC, DEG_PAD), jnp.float32),
    scratch_types=[
        pltpu.VMEM((CHUNK,), jnp.int32),
        pltpu.VMEM((CHUNK,), jnp.float32),
        pltpu.VMEM((DPT,), jnp.float32),
        pltpu.VMEM_SHARED((DEG_PAD,), jnp.float32),
    ],
)
def _deg_kernel(dst_hbm, out_hbm, dst_v, ones_v, zer_v, deg_sh):
    cid = lax.axis_index("c")
    sid = lax.axis_index("s")
    wid = cid * NS + sid
    for j in range(DPT // 16):
        zer_v[pl.ds(j * 16, 16)] = jnp.zeros((16,), jnp.float32)
    for j in range(CHUNK // 16):
        ones_v[pl.ds(j * 16, 16)] = jnp.ones((16,), jnp.float32)
    pltpu.sync_copy(zer_v, deg_sh.at[pl.ds(sid * DPT, DPT)])
    plsc.subcore_barrier()

    def step(i, carry):
        base = wid * EPT + i * CHUNK
        pltpu.sync_copy(dst_hbm.at[pl.ds(base, CHUNK)], dst_v)
        pltpu.sync_copy(ones_v, deg_sh.at[dst_v], add=True)
        return carry

    lax.fori_loop(0, NCHUNK, step, 0)
    plsc.subcore_barrier()
    pltpu.sync_copy(deg_sh.at[pl.ds(sid * DPT, DPT)],
                    out_hbm.at[cid, pl.ds(sid * DPT, DPT)])


def _transform_body(x_ref, w_ref, dp_ref, g_ref):
    h = jnp.dot(x_ref[...], w_ref[...], preferred_element_type=jnp.float32)
    deg = dp_ref[0, :N] + dp_ref[1, :N] + 1.0
    disq = lax.rsqrt(deg)
    g_ref[:N, :] = h * disq[:, None]
    g_ref[N:, :] = jnp.zeros((PAD_N - N, C), jnp.float32)


_transform = pl.pallas_call(
    _transform_body,
    out_shape=jax.ShapeDtypeStruct((PAD_N, C), jnp.float32),
)


@functools.partial(
    pl.kernel,
    mesh=_MESH,
    out_type=jax.ShapeDtypeStruct((NC, PAD_N, C), jnp.float32),
    scratch_types=[
        pltpu.VMEM((CHUNK,), jnp.int32),
        pltpu.VMEM((CHUNK,), jnp.int32),
        pltpu.VMEM((CHUNK, C), jnp.float32),
        pltpu.VMEM_SHARED((PAD_N, C), jnp.float32),
        pltpu.SemaphoreType.DMA,
    ],
)
def _agg_kernel(src_hbm, dst_hbm, g_hbm, out_hbm,
                src_v, dst_v, rows_v, acc_sh, sem):
    cid = lax.axis_index("c")
    sid = lax.axis_index("s")
    wid = cid * NS + sid
    pltpu.sync_copy(g_hbm.at[pl.ds(sid * RPT, RPT)],
                    acc_sh.at[pl.ds(sid * RPT, RPT)])
    plsc.subcore_barrier()

    def step(i, carry):
        base = wid * EPT + i * CHUNK
        pltpu.sync_copy(src_hbm.at[pl.ds(base, CHUNK)], src_v)
        pltpu.sync_copy(dst_hbm.at[pl.ds(base, CHUNK)], dst_v)
        pltpu.async_copy(g_hbm.at[src_v], rows_v, sem).wait()
        pltpu.sync_copy(rows_v, acc_sh.at[dst_v], add=True)
        return carry

    lax.fori_loop(0, NCHUNK, step, 0)
    plsc.subcore_barrier()
    pltpu.sync_copy(acc_sh.at[pl.ds(sid * RPT, RPT)],
                    out_hbm.at[cid, pl.ds(sid * RPT, RPT)])


def _finalize_body(s_ref, g_ref, dp_ref, b_ref, o_ref):
    deg = dp_ref[0, :N] + dp_ref[1, :N] + 1.0
    disq = lax.rsqrt(deg)
    tot = s_ref[0, :N, :] + s_ref[1, :N, :] - g_ref[:N, :]
    o_ref[...] = tot * disq[:, None] + b_ref[...][None, :]


_finalize = pl.pallas_call(
    _finalize_body,
    out_shape=jax.ShapeDtypeStruct((N, C), jnp.float32),
)


def kernel(x, edge_index, W, b):
    src = edge_index[0].astype(jnp.int32)
    dst = edge_index[1].astype(jnp.int32)
    pad = jnp.full((E_PAD - E,), N, jnp.int32)
    src_pad = jnp.concatenate([src, pad])
    dst_pad = jnp.concatenate([dst, pad])

    deg_parts = _deg_kernel(dst_pad)
    g_pad = _transform(x, W, deg_parts)
    s_parts = _agg_kernel(src_pad, dst_pad, g_pad)
    return _finalize(s_parts, g_pad, deg_parts, b)


# speedup vs baseline: 16.9520x; 1.3417x over previous
import functools

import jax
import jax.numpy as jnp
from jax import lax
from jax.experimental import pallas as pl
from jax.experimental.pallas import tpu as pltpu
from jax.experimental.pallas import tpu_sc as plsc

N = 10000
C = 128
E = 320000

NC = 2
NS = 16
NW = NC * NS

PAD_N = 10112
E_PAD = 327680
CHUNK = 128
EPT = E_PAD // NW
NCHUNK = EPT // CHUNK
RPT = PAD_N // NS
DEG_PAD = 10240
DPT = DEG_PAD // NS
NBUF = 2
DEG_Q = 8

_MESH = plsc.VectorSubcoreMesh(core_axis_name="c", subcore_axis_name="s")


@functools.partial(
    pl.kernel,
    mesh=_MESH,
    out_type=jax.ShapeDtypeStruct((NC, DEG_PAD), jnp.float32),
    scratch_types=[
        pltpu.VMEM((NCHUNK, CHUNK), jnp.int32),
        pltpu.VMEM((CHUNK,), jnp.float32),
        pltpu.VMEM((DPT,), jnp.float32),
        pltpu.VMEM_SHARED((DEG_PAD,), jnp.float32),
        pltpu.SemaphoreType.DMA,
    ],
)
def _deg_kernel(dst_hbm, out_hbm, dst_v, ones_v, zer_v, deg_sh, sem):
    cid = lax.axis_index("c")
    sid = lax.axis_index("s")
    wid = cid * NS + sid
    for j in range(DPT // 16):
        zer_v[pl.ds(j * 16, 16)] = jnp.zeros((16,), jnp.float32)
    for j in range(CHUNK // 16):
        ones_v[pl.ds(j * 16, 16)] = jnp.ones((16,), jnp.float32)
    pltpu.sync_copy(zer_v, deg_sh.at[pl.ds(sid * DPT, DPT)])
    pltpu.sync_copy(dst_hbm.at[wid], dst_v)
    plsc.subcore_barrier()

    def step(j, carry):
        for b in range(DEG_Q):
            pltpu.async_copy(ones_v, deg_sh.at[dst_v.at[j * DEG_Q + b]], sem,
                             add=True)
        for b in range(DEG_Q):
            pltpu.make_async_copy(ones_v, deg_sh.at[dst_v.at[j * DEG_Q + b]],
                                  sem).wait()
        return carry

    lax.fori_loop(0, NCHUNK // DEG_Q, step, 0)
    plsc.subcore_barrier()
    pltpu.sync_copy(deg_sh.at[pl.ds(sid * DPT, DPT)],
                    out_hbm.at[cid, pl.ds(sid * DPT, DPT)])


def _transform_body(x_ref, w_ref, dp_ref, g_ref):
    h = jnp.dot(x_ref[...], w_ref[...], preferred_element_type=jnp.float32)
    deg = dp_ref[0, :N] + dp_ref[1, :N] + 1.0
    disq = lax.rsqrt(deg)
    g_ref[:N, :] = h * disq[:, None]
    g_ref[N:, :] = jnp.zeros((PAD_N - N, C), jnp.float32)


_transform = pl.pallas_call(
    _transform_body,
    out_shape=jax.ShapeDtypeStruct((PAD_N, C), jnp.float32),
)


@functools.partial(
    pl.kernel,
    mesh=_MESH,
    out_type=jax.ShapeDtypeStruct((NC, PAD_N, C), jnp.float32),
    scratch_types=[
        pltpu.VMEM((NCHUNK, CHUNK), jnp.int32),
        pltpu.VMEM((NBUF, CHUNK), jnp.int32),
        pltpu.VMEM((NBUF, CHUNK, C), jnp.float32),
        pltpu.VMEM_SHARED((PAD_N, C), jnp.float32),
        pltpu.SemaphoreType.DMA,
        pltpu.SemaphoreType.DMA,
        pltpu.SemaphoreType.DMA,
        pltpu.SemaphoreType.DMA,
    ],
)
def _agg_kernel(src_hbm, dst_hbm, g_hbm, out_hbm,
                src_v, dstb_v, rows_v, acc_sh, sg0, sg1, sd0, sd1):
    sem_g = (sg0, sg1)
    sem_d = (sd0, sd1)
    cid = lax.axis_index("c")
    sid = lax.axis_index("s")
    wid = cid * NS + sid
    pltpu.sync_copy(g_hbm.at[pl.ds(sid * RPT, RPT)],
                    acc_sh.at[pl.ds(sid * RPT, RPT)])
    pltpu.sync_copy(src_hbm.at[wid], src_v)
    plsc.subcore_barrier()

    for b in range(NBUF):
        pltpu.async_copy(dst_hbm.at[wid, b], dstb_v.at[b], sem_d[b])
        pltpu.async_copy(g_hbm.at[src_v.at[b]], rows_v.at[b], sem_g[b])

    def step(j, carry):
        for b in range(NBUF):
            i = j * NBUF + b
            pltpu.make_async_copy(g_hbm.at[src_v.at[i]], rows_v.at[b],
                                  sem_g[b]).wait()
            pltpu.make_async_copy(dst_hbm.at[wid, i], dstb_v.at[b],
                                  sem_d[b]).wait()
            pltpu.sync_copy(rows_v.at[b], acc_sh.at[dstb_v.at[b]], add=True)
            nxt = i + NBUF

            @pl.when(nxt < NCHUNK)
            def _():
                pltpu.async_copy(dst_hbm.at[wid, nxt], dstb_v.at[b], sem_d[b])
                pltpu.async_copy(g_hbm.at[src_v.at[nxt]], rows_v.at[b],
                                 sem_g[b])
        return carry

    lax.fori_loop(0, NCHUNK // NBUF, step, 0)
    plsc.subcore_barrier()
    pltpu.sync_copy(acc_sh.at[pl.ds(sid * RPT, RPT)],
                    out_hbm.at[cid, pl.ds(sid * RPT, RPT)])


def _finalize_body(s_ref, g_ref, dp_ref, b_ref, o_ref):
    deg = dp_ref[0, :N] + dp_ref[1, :N] + 1.0
    disq = lax.rsqrt(deg)
    tot = s_ref[0, :N, :] + s_ref[1, :N, :] - g_ref[:N, :]
    o_ref[...] = tot * disq[:, None] + b_ref[...][None, :]


_finalize = pl.pallas_call(
    _finalize_body,
    out_shape=jax.ShapeDtypeStruct((N, C), jnp.float32),
)


def kernel(x, edge_index, W, b):
    src = edge_index[0].astype(jnp.int32)
    dst = edge_index[1].astype(jnp.int32)
    pad = jnp.full((E_PAD - E,), N, jnp.int32)
    src_rs = jnp.concatenate([src, pad]).reshape(NW, NCHUNK, CHUNK)
    dst_rs = jnp.concatenate([dst, pad]).reshape(NW, NCHUNK, CHUNK)

    deg_parts = _deg_kernel(dst_rs)
    g_pad = _transform(x, W, deg_parts)
    s_parts = _agg_kernel(src_rs, dst_rs, g_pad)
    return _finalize(s_parts, g_pad, deg_parts, b)
